# Initial kernel scaffold; baseline (speedup 1.0000x reference)
import jax
import jax.numpy as jnp
from jax.experimental import pallas as pl


def kernel(league_ids, table, W1, b1, W2, b2):
    raise NotImplementedError("write your pallas kernel here")


# speedup vs baseline: 65.7057x; 65.7057x over previous
import functools

import jax
import jax.numpy as jnp
from jax import lax
from jax.experimental import pallas as pl
from jax.experimental.pallas import tpu as pltpu
from jax.experimental.pallas import tpu_sc as plsc

_MAX_ID = 100000
_ROWS = _MAX_ID + 1
_LANE_BLK = 1024
_V_PAD = 98 * _LANE_BLK


def _mlp_body(tT_ref, w1_ref, b1_ref, w2_ref, b2_ref, out_ref):
    h = lax.dot_general(w1_ref[...], tT_ref[...], (((1,), (0,)), ((), ())),
                        preferred_element_type=jnp.float32)
    h = jnp.maximum(h + b1_ref[...][:, 0:1], 0.0)
    z = lax.dot_general(w2_ref[...], h, (((1,), (0,)), ((), ())),
                        preferred_element_type=jnp.float32)
    out_ref[...] = jax.nn.sigmoid(z + b2_ref[...][:, 0:1])


def _row_vals(table, W1, b1, W2, b2):
    tT = jnp.pad(table, ((0, _V_PAD - _ROWS), (0, 0))).T
    b1c = jnp.broadcast_to(b1[:, None], (8, 128))
    b2c = jnp.broadcast_to(b2[:, None], (1, 128))
    out = pl.pallas_call(
        _mlp_body,
        grid=(_V_PAD // _LANE_BLK,),
        in_specs=[
            pl.BlockSpec((16, _LANE_BLK), lambda i: (0, i)),
            pl.BlockSpec((8, 16), lambda i: (0, 0)),
            pl.BlockSpec((8, 128), lambda i: (0, 0)),
            pl.BlockSpec((1, 8), lambda i: (0, 0)),
            pl.BlockSpec((1, 128), lambda i: (0, 0)),
        ],
        out_specs=pl.BlockSpec((1, _LANE_BLK), lambda i: (0, i)),
        out_shape=jax.ShapeDtypeStruct((1, _V_PAD), jnp.float32),
    )(tT, W1, b1c, W2, b2c)
    return out.reshape(_V_PAD)


@functools.lru_cache(maxsize=None)
def _gather_kernel(total):
    info = plsc.get_sparse_core_info()
    nc, ns = info.num_cores, info.num_subcores
    nw = nc * ns
    per = total // nw
    chunk = 12800
    mesh = plsc.VectorSubcoreMesh(core_axis_name="c", subcore_axis_name="s")

    @functools.partial(
        pl.kernel, mesh=mesh,
        out_type=jax.ShapeDtypeStruct((total,), jnp.float32),
        compiler_params=pltpu.CompilerParams(needs_layout_passes=False),
        scratch_types=[
            pltpu.VMEM((_V_PAD,), jnp.float32),
            pltpu.VMEM((chunk,), jnp.int32),
            pltpu.VMEM((chunk,), jnp.float32),
        ],
    )
    def gather_k(vals_hbm, ids_hbm, out_hbm, vals_v, idx_v, out_v):
        wid = lax.axis_index("s") * nc + lax.axis_index("c")
        base = wid * per
        pltpu.sync_copy(vals_hbm, vals_v)

        def chunk_body(c, carry):
            off = base + c * chunk
            pltpu.sync_copy(ids_hbm.at[pl.ds(off, chunk)], idx_v)

            def step(j, carry2):
                ids16 = idx_v[pl.ds(j * 16, 16)]
                ids16 = jnp.minimum(jnp.maximum(ids16, 0), _MAX_ID)
                out_v[pl.ds(j * 16, 16)] = plsc.load_gather(vals_v, [ids16])
                return carry2

            lax.fori_loop(0, chunk // 16, step, 0)
            pltpu.sync_copy(out_v, out_hbm.at[pl.ds(off, chunk)])
            return carry

        lax.fori_loop(0, per // chunk, chunk_body, 0)

    return gather_k


def kernel(league_ids, table, W1, b1, W2, b2):
    vals = _row_vals(table, W1, b1, W2, b2)
    total = league_ids.size
    out = _gather_kernel(total)(vals, league_ids.reshape(total))
    return out.reshape(league_ids.shape)
